# baseline (speedup 1.0000x reference)
import functools

import jax
import jax.numpy as jnp
from jax import lax
from jax.experimental import pallas as pl
from jax.experimental.pallas import tpu as pltpu, tpu_sc as plsc

_NC, _NS, _L = 2, 16, 16
_D = 128
_DH = 64
_CHUNK = 128
_P = 4
_LA = 2
_GR = 16
_AGG_ROWS = 10240


def _stage1_body(u_ref, v_ref, g0_ref, g1_ref):
    for k in range(2):
        x = u_ref[...] if k == 0 else v_ref[...]
        col = lax.broadcasted_iota(jnp.int32, x.shape, 1)
        xm = jnp.where(col == 0, 0.0, x)
        s = jnp.sum(xm * xm, axis=1, keepdims=True)
        nrm = jnp.sqrt(jnp.clip(s, 1e-15, None))
        x0 = jnp.maximum(jnp.sqrt(1.0 + s), 1.0 + 1e-7)
        theta = jnp.log(x0 + jnp.sqrt(x0 * x0 - 1.0))
        g = jnp.where(col == 0, 1.0, xm * (theta / nrm))
        g0_ref[k] = g[:, :_DH]
        g1_ref[k] = g[:, _DH:]


def _stage3_body(p_ref, w_ref, b_ref, y_ref):
    a = jnp.concatenate([p_ref[0], p_ref[1]], axis=1)
    deg = jnp.maximum(a[:, 0:1], 1.0)
    agg = a / deg
    colw = lax.broadcasted_iota(jnp.int32, w_ref.shape, 1)
    wm = jnp.where(colw == 0, 0.0, w_ref[...])
    v = lax.dot_general(agg, wm, (((1,), (1,)), ((), ())),
                        preferred_element_type=jnp.float32) + b_ref[...]
    col = lax.broadcasted_iota(jnp.int32, v.shape, 1)
    vm = jnp.where(col == 0, 0.0, v)
    s2 = jnp.sum(vm * vm, axis=1, keepdims=True)
    nrm2 = jnp.sqrt(jnp.clip(s2, 1e-15, None))
    e = jnp.exp(nrm2)
    r = (e - 1.0 / e) * 0.5 / nrm2
    xr = vm * r
    y0 = jnp.sqrt(1.0 + jnp.sum(xr * xr, axis=1, keepdims=True))
    y_ref[...] = jnp.where(col == 0, y0, xr)


def _sc_body(n_nodes, g0_hbm, g1_hbm, src_hbm, dst_hbm, out_hbm,
             idx_s_v, idx_d_v, rows_v, agg_sp, tbl_sp, sem, sem2, sem_i, sem_z):
    c = lax.axis_index("c")
    s = lax.axis_index("s")
    g_rows = src_hbm.shape[0] // _NS
    per_tile = _AGG_ROWS // _NS

    def idx_start(gi, buf):
        pltpu.async_copy(src_hbm.at[pl.ds(s * g_rows + gi * _GR, _GR)],
                         idx_s_v.at[buf], sem_i.at[buf])
        pltpu.async_copy(dst_hbm.at[pl.ds(s * g_rows + gi * _GR, _GR)],
                         idx_d_v.at[buf], sem_i.at[buf])

    def idx_wait(buf):
        pltpu.make_async_copy(src_hbm.at[pl.ds(0, _GR)], idx_s_v.at[buf],
                              sem_i.at[buf]).wait()
        pltpu.make_async_copy(dst_hbm.at[pl.ds(0, _GR)], idx_d_v.at[buf],
                              sem_i.at[buf]).wait()

    def gather(idxbuf, k, t):
        if k % 2 == 0:
            @pl.when(c == 0)
            def _g0():
                pltpu.async_copy(g0_hbm.at[idx_s_v.at[idxbuf, k]],
                                 rows_v.at[t], sem.at[t])

            @pl.when(c != 0)
            def _g1():
                pltpu.async_copy(g1_hbm.at[idx_s_v.at[idxbuf, k]],
                                 rows_v.at[t], sem.at[t])
        else:
            pltpu.async_copy(tbl_sp.at[idx_s_v.at[idxbuf, k]],
                             rows_v.at[t], sem.at[t])

    def gather_wait(t):
        pltpu.make_async_copy(g0_hbm.at[pl.ds(0, _CHUNK)], rows_v.at[t],
                              sem.at[t]).wait()

    def scatter(idxbuf, k, t):
        pltpu.async_copy(rows_v.at[t], agg_sp.at[idx_d_v.at[idxbuf, k]],
                         sem2.at[t], add=True)

    def scatter_wait(t):
        pltpu.make_async_copy(rows_v.at[t], agg_sp.at[pl.ds(0, _CHUNK)],
                              sem2.at[t]).wait()

    def zbody(i, carry):
        for jj in range(_DH // 16):
            rows_v[0, i, pl.ds(jj * 16, 16)] = jnp.zeros((16,), jnp.float32)
        return carry
    lax.fori_loop(0, _CHUNK, zbody, 0)
    for k in range(per_tile // _CHUNK):
        pltpu.async_copy(rows_v.at[0],
                         agg_sp.at[pl.ds(s * per_tile + k * _CHUNK, _CHUNK)],
                         sem_z)
    @pl.when(s < _NS - 1)
    def _tbl_main():
        @pl.when(c == 0)
        def _t0():
            pltpu.async_copy(g0_hbm.at[pl.ds(s * 640, 640)],
                             tbl_sp.at[pl.ds(s * 640, 640)], sem_z)

        @pl.when(c != 0)
        def _t1():
            pltpu.async_copy(g1_hbm.at[pl.ds(s * 640, 640)],
                             tbl_sp.at[pl.ds(s * 640, 640)], sem_z)

    @pl.when(s == _NS - 1)
    def _tbl_tail():
        last = (_NS - 1) * 640
        @pl.when(c == 0)
        def _t0():
            pltpu.async_copy(g0_hbm.at[pl.ds(last, n_nodes - last)],
                             tbl_sp.at[pl.ds(last, n_nodes - last)], sem_z)

        @pl.when(c != 0)
        def _t1():
            pltpu.async_copy(g1_hbm.at[pl.ds(last, n_nodes - last)],
                             tbl_sp.at[pl.ds(last, n_nodes - last)], sem_z)

    idx_start(0, 0)
    for k in range(per_tile // _CHUNK):
        pltpu.make_async_copy(rows_v.at[0],
                              agg_sp.at[pl.ds(0, _CHUNK)], sem_z).wait()

    @pl.when(s < _NS - 1)
    def _tbl_main_wait():
        pltpu.make_async_copy(g0_hbm.at[pl.ds(0, 640)],
                              tbl_sp.at[pl.ds(0, 640)], sem_z).wait()

    @pl.when(s == _NS - 1)
    def _tbl_tail_wait():
        last = (_NS - 1) * 640
        pltpu.make_async_copy(g0_hbm.at[pl.ds(0, n_nodes - last)],
                              tbl_sp.at[pl.ds(0, n_nodes - last)], sem_z).wait()

    plsc.subcore_barrier()
    idx_wait(0)

    for k in range(_LA):
        gather(0, k, k)

    ng = g_rows // _GR

    def group_body(gi, carry):
        buf = lax.rem(gi, 2)
        nbuf = lax.rem(gi + 1, 2)
        for k in range(_GR):
            t = k % _P
            t4 = (k + _LA) % _P
            gather_wait(t)
            scatter(buf, k, t)
            if k == _LA:
                @pl.when(gi + 1 < ng)
                def _prefetch_idx():
                    idx_start(gi + 1, nbuf)
            if k == _GR - _LA:
                @pl.when(gi + 1 < ng)
                def _wait_idx():
                    idx_wait(nbuf)
            if k < _LA:
                @pl.when(gi > 0)
                def _wait_prev_scatter():
                    scatter_wait(t4)
            else:
                scatter_wait(t4)
            if k < _GR - _LA:
                gather(buf, k + _LA, t4)
            else:
                @pl.when(gi + 1 < ng)
                def _gather_next_group():
                    gather(nbuf, k + _LA - _GR, t4)
        return carry
    lax.fori_loop(0, ng, group_body, 0)

    for k in range(_GR - _LA, _GR):
        scatter_wait(k % _P)
    plsc.subcore_barrier()

    @pl.when(s < _NS - 1)
    def _copy_main():
        pltpu.sync_copy(agg_sp.at[pl.ds(s * 640, 640)],
                        out_hbm.at[c, pl.ds(s * 640, 640)])

    @pl.when(s == _NS - 1)
    def _copy_tail():
        last = (_NS - 1) * 640
        pltpu.sync_copy(agg_sp.at[pl.ds(last, n_nodes - last)],
                        out_hbm.at[c, pl.ds(last, n_nodes - last)])


def kernel(edge_index, utg_weight, vtg_weight, W, b):
    n = utg_weight.shape[0] + vtg_weight.shape[0]
    e = edge_index.shape[1]
    d = utg_weight.shape[1]

    nh = n // 2
    nblk = 5
    blk = nh // nblk
    g0, g1 = pl.pallas_call(
        _stage1_body,
        grid=(nblk,),
        in_specs=[pl.BlockSpec((blk, d), lambda i: (i, 0)),
                  pl.BlockSpec((blk, d), lambda i: (i, 0))],
        out_specs=[pl.BlockSpec((2, blk, _DH), lambda i: (0, i, 0)),
                   pl.BlockSpec((2, blk, _DH), lambda i: (0, i, 0))],
        out_shape=[jax.ShapeDtypeStruct((2, nh, _DH), jnp.float32),
                   jax.ShapeDtypeStruct((2, nh, _DH), jnp.float32)],
    )(utg_weight, vtg_weight)
    g0 = g0.reshape(n, _DH)
    g1 = g1.reshape(n, _DH)

    g_rows = -(-e // (_NS * _CHUNK * 8)) * 8
    ep = _NS * g_rows * _CHUNK
    pad_ar = jnp.arange(ep - e, dtype=jnp.int32)
    srcp = jnp.concatenate(
        [edge_index[0], pad_ar % n]).reshape(_NS * g_rows, _CHUNK)
    dstp = jnp.concatenate(
        [edge_index[1], n + pad_ar % (_AGG_ROWS - n)]).reshape(_NS * g_rows, _CHUNK)

    mesh = plsc.VectorSubcoreMesh(core_axis_name="c", subcore_axis_name="s")
    partial = pl.kernel(
        functools.partial(_sc_body, n),
        out_type=jax.ShapeDtypeStruct((_NC, n, _DH), jnp.float32),
        mesh=mesh,
        compiler_params=pltpu.CompilerParams(use_tc_tiling_on_sc=False),
        scratch_types=[
            pltpu.VMEM((2, _GR, _CHUNK), jnp.int32),
            pltpu.VMEM((2, _GR, _CHUNK), jnp.int32),
            pltpu.VMEM((_P, _CHUNK, _DH), jnp.float32),
            pltpu.VMEM_SHARED((_AGG_ROWS, _DH), jnp.float32),
            pltpu.VMEM_SHARED((10000, _DH), jnp.float32),
            pltpu.SemaphoreType.DMA((_P,)),
            pltpu.SemaphoreType.DMA((_P,)),
            pltpu.SemaphoreType.DMA((2,)),
            pltpu.SemaphoreType.DMA,
        ],
    )(g0, g1, srcp, dstp)

    nblk3 = 10
    y = pl.pallas_call(
        _stage3_body,
        grid=(nblk3,),
        in_specs=[pl.BlockSpec((_NC, n // nblk3, _DH), lambda i: (0, i, 0)),
                  pl.BlockSpec((d, d), lambda i: (0, 0)),
                  pl.BlockSpec((1, d), lambda i: (0, 0))],
        out_specs=pl.BlockSpec((n // nblk3, d), lambda i: (i, 0)),
        out_shape=jax.ShapeDtypeStruct((n, d), jnp.float32),
    )(partial, W, b.reshape(1, d))
    return y


# speedup vs baseline: 1.2891x; 1.2891x over previous
import functools

import jax
import jax.numpy as jnp
from jax import lax
from jax.experimental import pallas as pl
from jax.experimental.pallas import tpu as pltpu, tpu_sc as plsc

_NC, _NS, _L = 2, 16, 16
_D = 128
_DH = 64
_CHUNK = 128
_P = 8
_LA = 4
_GR = 16
_AGG_ROWS = 10240


def _stage1_body(u_ref, v_ref, g0_ref, g1_ref):
    for k in range(2):
        x = u_ref[...] if k == 0 else v_ref[...]
        col = lax.broadcasted_iota(jnp.int32, x.shape, 1)
        xm = jnp.where(col == 0, 0.0, x)
        s = jnp.sum(xm * xm, axis=1, keepdims=True)
        nrm = jnp.sqrt(jnp.clip(s, 1e-15, None))
        x0 = jnp.maximum(jnp.sqrt(1.0 + s), 1.0 + 1e-7)
        theta = jnp.log(x0 + jnp.sqrt(x0 * x0 - 1.0))
        g = jnp.where(col == 0, 1.0, xm * (theta / nrm))
        g0_ref[k] = g[:, :_DH]
        g1_ref[k] = g[:, _DH:]


def _stage3_body(p_ref, w_ref, b_ref, y_ref):
    a = jnp.concatenate([p_ref[0], p_ref[1]], axis=1)
    deg = jnp.maximum(a[:, 0:1], 1.0)
    agg = a / deg
    colw = lax.broadcasted_iota(jnp.int32, w_ref.shape, 1)
    wm = jnp.where(colw == 0, 0.0, w_ref[...])
    v = lax.dot_general(agg, wm, (((1,), (1,)), ((), ())),
                        preferred_element_type=jnp.float32) + b_ref[...]
    col = lax.broadcasted_iota(jnp.int32, v.shape, 1)
    vm = jnp.where(col == 0, 0.0, v)
    s2 = jnp.sum(vm * vm, axis=1, keepdims=True)
    nrm2 = jnp.sqrt(jnp.clip(s2, 1e-15, None))
    e = jnp.exp(nrm2)
    r = (e - 1.0 / e) * 0.5 / nrm2
    xr = vm * r
    y0 = jnp.sqrt(1.0 + jnp.sum(xr * xr, axis=1, keepdims=True))
    y_ref[...] = jnp.where(col == 0, y0, xr)


def _sc_body(n_nodes, g0_hbm, g1_hbm, src_hbm, dst_hbm, out_hbm,
             idx_s_v, idx_d_v, rows_v, agg_sp, sem, sem2, sem_i, sem_z):
    c = lax.axis_index("c")
    s = lax.axis_index("s")
    g_rows = src_hbm.shape[0] // _NS
    per_tile = _AGG_ROWS // _NS

    def idx_start(gi, buf):
        pltpu.async_copy(src_hbm.at[pl.ds(s * g_rows + gi * _GR, _GR)],
                         idx_s_v.at[buf], sem_i.at[buf])
        pltpu.async_copy(dst_hbm.at[pl.ds(s * g_rows + gi * _GR, _GR)],
                         idx_d_v.at[buf], sem_i.at[buf])

    def idx_wait(buf):
        pltpu.make_async_copy(src_hbm.at[pl.ds(0, _GR)], idx_s_v.at[buf],
                              sem_i.at[buf]).wait()
        pltpu.make_async_copy(dst_hbm.at[pl.ds(0, _GR)], idx_d_v.at[buf],
                              sem_i.at[buf]).wait()

    def gather(idxbuf, k, t):
        @pl.when(c == 0)
        def _g0():
            pltpu.async_copy(g0_hbm.at[idx_s_v.at[idxbuf, k]], rows_v.at[t],
                             sem.at[t])

        @pl.when(c != 0)
        def _g1():
            pltpu.async_copy(g1_hbm.at[idx_s_v.at[idxbuf, k]], rows_v.at[t],
                             sem.at[t])

    def gather_wait(t):
        pltpu.make_async_copy(g0_hbm.at[pl.ds(0, _CHUNK)], rows_v.at[t],
                              sem.at[t]).wait()

    def scatter(idxbuf, k, t):
        pltpu.async_copy(rows_v.at[t], agg_sp.at[idx_d_v.at[idxbuf, k]],
                         sem2.at[t], add=True)

    def scatter_wait(t):
        pltpu.make_async_copy(rows_v.at[t], agg_sp.at[pl.ds(0, _CHUNK)],
                              sem2.at[t]).wait()

    def zbody(i, carry):
        for jj in range(_DH // 16):
            rows_v[0, i, pl.ds(jj * 16, 16)] = jnp.zeros((16,), jnp.float32)
        return carry
    lax.fori_loop(0, _CHUNK, zbody, 0)
    for k in range(per_tile // _CHUNK):
        pltpu.async_copy(rows_v.at[0],
                         agg_sp.at[pl.ds(s * per_tile + k * _CHUNK, _CHUNK)],
                         sem_z)
    idx_start(0, 0)
    for k in range(per_tile // _CHUNK):
        pltpu.make_async_copy(rows_v.at[0],
                              agg_sp.at[pl.ds(0, _CHUNK)], sem_z).wait()
    plsc.subcore_barrier()
    idx_wait(0)

    for k in range(_LA):
        gather(0, k, k)

    ng = g_rows // _GR

    def group_body(gi, carry):
        buf = lax.rem(gi, 2)
        nbuf = lax.rem(gi + 1, 2)
        for k in range(_GR):
            t = k % _P
            t4 = (k + _LA) % _P
            gather_wait(t)
            scatter(buf, k, t)
            if k == _LA:
                @pl.when(gi + 1 < ng)
                def _prefetch_idx():
                    idx_start(gi + 1, nbuf)
            if k == _GR - _LA:
                @pl.when(gi + 1 < ng)
                def _wait_idx():
                    idx_wait(nbuf)
            if k < _LA:
                @pl.when(gi > 0)
                def _wait_prev_scatter():
                    scatter_wait(t4)
            else:
                scatter_wait(t4)
            if k < _GR - _LA:
                gather(buf, k + _LA, t4)
            else:
                @pl.when(gi + 1 < ng)
                def _gather_next_group():
                    gather(nbuf, k + _LA - _GR, t4)
        return carry
    lax.fori_loop(0, ng, group_body, 0)

    for k in range(_GR - _LA, _GR):
        scatter_wait(k % _P)
    plsc.subcore_barrier()

    @pl.when(s < _NS - 1)
    def _copy_main():
        pltpu.sync_copy(agg_sp.at[pl.ds(s * 640, 640)],
                        out_hbm.at[c, pl.ds(s * 640, 640)])

    @pl.when(s == _NS - 1)
    def _copy_tail():
        last = (_NS - 1) * 640
        pltpu.sync_copy(agg_sp.at[pl.ds(last, n_nodes - last)],
                        out_hbm.at[c, pl.ds(last, n_nodes - last)])


def kernel(edge_index, utg_weight, vtg_weight, W, b):
    n = utg_weight.shape[0] + vtg_weight.shape[0]
    e = edge_index.shape[1]
    d = utg_weight.shape[1]

    nh = n // 2
    nblk = 5
    blk = nh // nblk
    g0, g1 = pl.pallas_call(
        _stage1_body,
        grid=(nblk,),
        in_specs=[pl.BlockSpec((blk, d), lambda i: (i, 0)),
                  pl.BlockSpec((blk, d), lambda i: (i, 0))],
        out_specs=[pl.BlockSpec((2, blk, _DH), lambda i: (0, i, 0)),
                   pl.BlockSpec((2, blk, _DH), lambda i: (0, i, 0))],
        out_shape=[jax.ShapeDtypeStruct((2, nh, _DH), jnp.float32),
                   jax.ShapeDtypeStruct((2, nh, _DH), jnp.float32)],
    )(utg_weight, vtg_weight)
    g0 = g0.reshape(n, _DH)
    g1 = g1.reshape(n, _DH)

    g_rows = -(-e // (_NS * _CHUNK * 8)) * 8
    ep = _NS * g_rows * _CHUNK
    pad_ar = jnp.arange(ep - e, dtype=jnp.int32)
    srcp = jnp.concatenate(
        [edge_index[0], pad_ar % n]).reshape(_NS * g_rows, _CHUNK)
    dstp = jnp.concatenate(
        [edge_index[1], n + pad_ar % (_AGG_ROWS - n)]).reshape(_NS * g_rows, _CHUNK)

    mesh = plsc.VectorSubcoreMesh(core_axis_name="c", subcore_axis_name="s")
    partial = pl.kernel(
        functools.partial(_sc_body, n),
        out_type=jax.ShapeDtypeStruct((_NC, n, _DH), jnp.float32),
        mesh=mesh,
        compiler_params=pltpu.CompilerParams(use_tc_tiling_on_sc=False),
        scratch_types=[
            pltpu.VMEM((2, _GR, _CHUNK), jnp.int32),
            pltpu.VMEM((2, _GR, _CHUNK), jnp.int32),
            pltpu.VMEM((_P, _CHUNK, _DH), jnp.float32),
            pltpu.VMEM_SHARED((_AGG_ROWS, _DH), jnp.float32),
            pltpu.SemaphoreType.DMA((_P,)),
            pltpu.SemaphoreType.DMA((_P,)),
            pltpu.SemaphoreType.DMA((2,)),
            pltpu.SemaphoreType.DMA,
        ],
    )(g0, g1, srcp, dstp)

    nblk3 = 5
    y = pl.pallas_call(
        _stage3_body,
        grid=(nblk3,),
        in_specs=[pl.BlockSpec((_NC, n // nblk3, _DH), lambda i: (0, i, 0)),
                  pl.BlockSpec((d, d), lambda i: (0, 0)),
                  pl.BlockSpec((1, d), lambda i: (0, 0))],
        out_specs=pl.BlockSpec((n // nblk3, d), lambda i: (i, 0)),
        out_shape=jax.ShapeDtypeStruct((n, d), jnp.float32),
    )(partial, W, b.reshape(1, d))
    return y
